# baseline (speedup 1.0000x reference)
import functools

import jax
import jax.numpy as jnp
from jax import lax
from jax.experimental import pallas as pl
from jax.experimental.pallas import tpu as pltpu
from jax.experimental.pallas import tpu_sc as plsc

_NC = 2
_NS = 16
_NW = _NC * _NS
_L = 16
_K = 16
_NBUF = 3


@functools.cache
def _emb_call(n_batch: int, n_seq: int, d: int):
    wpb = _NW // n_batch
    spw = n_seq // wpb
    nchunk = spw // _K
    mesh = plsc.VectorSubcoreMesh(core_axis_name="c", subcore_axis_name="s")

    @functools.partial(
        pl.kernel,
        mesh=mesh,
        out_type=jax.ShapeDtypeStruct((n_seq, n_batch, d), jnp.float32),
        scratch_types=[
            pltpu.VMEM((spw,), jnp.int32),
            pltpu.VMEM((spw,), jnp.int32),
            pltpu.VMEM((_NBUF, _K, d), jnp.float32),
            pltpu.VMEM((_NBUF, _K, d), jnp.float32),
        ] + [pltpu.SemaphoreType.DMA] * (2 * _NBUF),
    )
    def k(tok_hbm, pos_hbm, wt_hbm, pt_hbm, out_hbm, tok_v, pos_v, wbuf,
          pbuf, *sems):
        gsem = sems[:_NBUF]
        osem = sems[_NBUF:]
        wid = lax.axis_index("s") * _NC + lax.axis_index("c")
        b = wid // wpb
        s0w = (wid % wpb) * spw
        ci = pltpu.async_copy(tok_hbm.at[b, pl.ds(s0w, spw)], tok_v,
                              sems[0])
        cj = pltpu.async_copy(pos_hbm.at[b, pl.ds(s0w, spw)], pos_v,
                              sems[0])
        ci.wait()
        cj.wait()

        def gather_desc(g, s):
            off = g * _K
            cw = pltpu.make_async_copy(wt_hbm.at[tok_v.at[pl.ds(off, _K)]],
                                       wbuf.at[s], gsem[s])
            cp = pltpu.make_async_copy(pt_hbm.at[pos_v.at[pl.ds(off, _K)]],
                                       pbuf.at[s], gsem[s])
            return cw, cp

        def out_desc(g, s):
            return pltpu.make_async_copy(
                wbuf.at[s], out_hbm.at[pl.ds(s0w + g * _K, _K), b], osem[s])

        for s in range(_NBUF - 1):
            cw, cp = gather_desc(s, s)
            cw.start()
            cp.start()

        def block_body(blk, carry):
            for s in range(_NBUF):
                h = blk * _NBUF + s
                cw, cp = gather_desc(h, s)
                cw.wait()
                cp.wait()

                nxt = h + _NBUF - 1
                s2 = (s + _NBUF - 1) % _NBUF

                @pl.when(jnp.logical_and(h >= 1, nxt < nchunk))
                def _():
                    out_desc(h - 1, s2).wait()

                @pl.when(nxt < nchunk)
                def _():
                    cw2, cp2 = gather_desc(nxt, s2)
                    cw2.start()
                    cp2.start()

                def row_body(r, c2, s=s):
                    for c in range(d // _L):
                        sl = pl.ds(c * _L, _L)
                        plsc.addupdate(wbuf.at[s, r, sl], pbuf[s, r, sl])
                    return c2

                lax.fori_loop(0, _K, row_body, 0)
                out_desc(h, s).start()
            return carry

        nmain = nchunk - nchunk % _NBUF
        lax.fori_loop(0, nmain // _NBUF, block_body, 0)
        for g in range(nmain, nchunk):
            sr = g % _NBUF
            cw, cp = gather_desc(g, sr)
            cw.wait()
            cp.wait()

            def row_body(r, c2, sr=sr):
                for c in range(d // _L):
                    sl = pl.ds(c * _L, _L)
                    plsc.addupdate(wbuf.at[sr, r, sl], pbuf[sr, r, sl])
                return c2

            lax.fori_loop(0, _K, row_body, 0)
            out_desc(g, sr).start()
        for j in range(_NBUF):
            g = nchunk - _NBUF + j
            out_desc(g, g % _NBUF).wait()

    return k


def kernel(tokens, position_ids, word_table, pos_table):
    b, s = tokens.shape
    d = word_table.shape[1]
    tok = tokens.astype(jnp.int32)
    pos = position_ids.astype(jnp.int32)
    return _emb_call(b, s, d)(tok, pos, word_table, pos_table)


# speedup vs baseline: 1.7754x; 1.7754x over previous
import functools

import jax
import jax.numpy as jnp
from jax import lax
from jax.experimental import pallas as pl
from jax.experimental.pallas import tpu as pltpu
from jax.experimental.pallas import tpu_sc as plsc

_NC = 2
_NS = 16
_NW = _NC * _NS
_L = 16
_K = 8
_NBUF = 4


@functools.cache
def _emb_call(n_batch: int, n_seq: int, d: int):
    wpb = _NW // n_batch
    spw = n_seq // wpb
    nchunk = spw // _K
    mesh = plsc.VectorSubcoreMesh(core_axis_name="c", subcore_axis_name="s")

    @functools.partial(
        pl.kernel,
        mesh=mesh,
        out_type=jax.ShapeDtypeStruct((n_seq, n_batch, d), jnp.float32),
        scratch_types=[
            pltpu.VMEM((spw,), jnp.int32),
            pltpu.VMEM((spw,), jnp.int32),
            pltpu.VMEM((_NBUF, _K, d), jnp.float32),
            pltpu.VMEM((_NBUF, _K, d), jnp.float32),
        ] + [pltpu.SemaphoreType.DMA] * (2 * _NBUF),
    )
    def k(tok_hbm, pos_hbm, wt_hbm, pt_hbm, out_hbm, tok_v, pos_v, wbuf,
          pbuf, *sems):
        gsem = sems[:_NBUF]
        osem = sems[_NBUF:]
        wid = lax.axis_index("s") * _NC + lax.axis_index("c")
        b = wid // wpb
        s0w = (wid % wpb) * spw
        ci = pltpu.async_copy(tok_hbm.at[b, pl.ds(s0w, spw)], tok_v,
                              sems[0])
        cj = pltpu.async_copy(pos_hbm.at[b, pl.ds(s0w, spw)], pos_v,
                              sems[0])
        ci.wait()
        cj.wait()

        def gather_desc(g, s):
            off = g * _K
            cw = pltpu.make_async_copy(wt_hbm.at[tok_v.at[pl.ds(off, _K)]],
                                       wbuf.at[s], gsem[s])
            cp = pltpu.make_async_copy(pt_hbm.at[pos_v.at[pl.ds(off, _K)]],
                                       pbuf.at[s], gsem[s])
            return cw, cp

        def out_desc(g, s):
            return pltpu.make_async_copy(
                wbuf.at[s], out_hbm.at[pl.ds(s0w + g * _K, _K), b], osem[s])

        for s in range(_NBUF - 1):
            cw, cp = gather_desc(s, s)
            cw.start()
            cp.start()

        def block_body(blk, carry):
            for s in range(_NBUF):
                h = blk * _NBUF + s
                cw, cp = gather_desc(h, s)
                cw.wait()
                cp.wait()

                nxt = h + _NBUF - 1
                s2 = (s + _NBUF - 1) % _NBUF

                @pl.when(jnp.logical_and(h >= 1, nxt < nchunk))
                def _():
                    out_desc(h - 1, s2).wait()

                @pl.when(nxt < nchunk)
                def _():
                    cw2, cp2 = gather_desc(nxt, s2)
                    cw2.start()
                    cp2.start()

                def row_body(r, c2, s=s):
                    for c in range(d // _L):
                        sl = pl.ds(c * _L, _L)
                        plsc.addupdate(wbuf.at[s, r, sl], pbuf[s, r, sl])
                    return c2

                lax.fori_loop(0, _K, row_body, 0)
                out_desc(h, s).start()
            return carry

        nmain = nchunk - nchunk % _NBUF
        lax.fori_loop(0, nmain // _NBUF, block_body, 0)
        for g in range(nmain, nchunk):
            sr = g % _NBUF
            cw, cp = gather_desc(g, sr)
            cw.wait()
            cp.wait()

            def row_body(r, c2, sr=sr):
                for c in range(d // _L):
                    sl = pl.ds(c * _L, _L)
                    plsc.addupdate(wbuf.at[sr, r, sl], pbuf[sr, r, sl])
                return c2

            lax.fori_loop(0, _K, row_body, 0)
            out_desc(g, sr).start()
        for j in range(_NBUF):
            g = nchunk - _NBUF + j
            out_desc(g, g % _NBUF).wait()

    return k


def kernel(tokens, position_ids, word_table, pos_table):
    b, s = tokens.shape
    d = word_table.shape[1]
    tok = tokens.astype(jnp.int32)
    pos = position_ids.astype(jnp.int32)
    return _emb_call(b, s, d)(tok, pos, word_table, pos_table)
